# Initial kernel scaffold; baseline (speedup 1.0000x reference)
import jax
import jax.numpy as jnp
from jax.experimental import pallas as pl


def kernel(feat, edge_index, W_enc1, b_enc1, W_enc2, b_enc2, W_conv1, b_conv1, W_conv2, b_conv2, bn1_g, bn1_b, bn2_g, bn2_b, W_dec1, b_dec1, W_dec2, b_dec2):
    raise NotImplementedError("write your pallas kernel here")


# speedup vs baseline: 6.6469x; 6.6469x over previous
import functools

import jax
import jax.numpy as jnp
from jax import lax
from jax.experimental import pallas as pl
from jax.experimental.pallas import tpu as pltpu
from jax.experimental.pallas import tpu_sc as plsc

N = 10000
E = 320000
D_IN = 128
H1 = 32
H2 = 64
G1 = 128
G2 = 256
PH = 32

NC = 2
NS = 16
NW = NC * NS
K = 80
EPT = E // NS
CHUNKS = EPT // K
NPAIRS = CHUNKS // 2
KV = K // 16
N_PAD = 10240
ROWS_PT = N_PAD // NS
DW = 128
HALF = 5120
TRASH = HALF
TBL = 5248
ZRO_PT = TBL // NS
OUT_PT = HALF // NS


@functools.lru_cache(maxsize=None)
def _mesh():
    return plsc.VectorSubcoreMesh(core_axis_name="c", subcore_axis_name="s",
                                  num_cores=NC, num_subcores=NS)


DEG_ROWS = N_PAD // DW
DEG_CPT = E // NW // K


def _deg_body(src_hbm, rowid_hbm, zeros_hbm, out_hbm,
              idx_v, rowid_v, hist_v, tbl_sh):
    c = lax.axis_index("c")
    s = lax.axis_index("s")
    wid = c * NS + s
    pltpu.sync_copy(src_hbm.at[wid], idx_v)
    pltpu.sync_copy(rowid_hbm, rowid_v)

    @pl.when(s == 0)
    def _():
        pltpu.sync_copy(zeros_hbm, tbl_sh)

    def zero(r, carry):
        for v in range(DW // 16):
            hist_v[r, pl.ds(v * 16, 16)] = jnp.zeros((16,), jnp.float32)
        return carry

    lax.fori_loop(0, DEG_ROWS, zero, 0)

    ones16 = jnp.ones((16,), jnp.float32)

    def count(r, carry):
        for v in range(KV):
            iv = idx_v[r, pl.ds(v * 16, 16)]
            rv = lax.shift_right_logical(iv, jnp.int32(7))
            cv = iv & jnp.int32(127)
            plsc.addupdate_scatter(hist_v, [rv, cv], ones16)
        return carry

    lax.fori_loop(0, DEG_CPT, count, 0)
    plsc.subcore_barrier()
    pltpu.sync_copy(hist_v, tbl_sh.at[rowid_v.at[0]], add=True)
    plsc.subcore_barrier()

    @pl.when(s == 0)
    def _():
        pltpu.sync_copy(tbl_sh, out_hbm.at[c])


@functools.lru_cache(maxsize=None)
def _deg_call():
    return pl.kernel(
        _deg_body,
        out_type=jax.ShapeDtypeStruct((NC, DEG_ROWS, DW), jnp.float32),
        mesh=_mesh(),
        scratch_types=[
            pltpu.VMEM((DEG_CPT, K), jnp.int32),
            pltpu.VMEM((1, DEG_ROWS), jnp.int32),
            pltpu.VMEM((DEG_ROWS, DW), jnp.float32),
            pltpu.VMEM_SHARED((DEG_ROWS, DW), jnp.float32),
        ],
        compiler_params=pltpu.CompilerParams(needs_layout_passes=False),
    )


def _agg_body(x_hbm, src_hbm, dst_hbm, zeros_hbm, out_hbm,
              sidx_v, didx_v, msg_a, msg_b, sem_a, sem_b, agg_sh):
    c = lax.axis_index("c")
    s = lax.axis_index("s")
    pltpu.sync_copy(zeros_hbm, agg_sh.at[pl.ds(s * ZRO_PT, ZRO_PT)])
    pltpu.sync_copy(src_hbm.at[s], sidx_v)
    pltpu.sync_copy(dst_hbm.at[s], didx_v)

    base = c * HALF

    def rewrite(r, carry):
        for v in range(KV):
            dv = didx_v[r, pl.ds(v * 16, 16)]
            t = dv - base
            ok = (t >= 0) & (t < HALF)
            didx_v[r, pl.ds(v * 16, 16)] = jnp.where(ok, t, TRASH)
        return carry

    lax.fori_loop(0, CHUNKS, rewrite, 0)
    plsc.subcore_barrier()

    pltpu.async_copy(x_hbm.at[sidx_v.at[0]], msg_a, sem_a)

    def pair(i, carry):
        a = 2 * i
        pltpu.async_copy(x_hbm.at[sidx_v.at[a + 1]], msg_b, sem_b)
        pltpu.make_async_copy(x_hbm.at[sidx_v.at[a]], msg_a, sem_a).wait()
        pltpu.sync_copy(msg_a, agg_sh.at[didx_v.at[a]], add=True)

        @pl.when(i < NPAIRS - 1)
        def _():
            pltpu.async_copy(x_hbm.at[sidx_v.at[a + 2]], msg_a, sem_a)

        pltpu.make_async_copy(x_hbm.at[sidx_v.at[a + 1]], msg_b, sem_b).wait()
        pltpu.sync_copy(msg_b, agg_sh.at[didx_v.at[a + 1]], add=True)
        return carry

    lax.fori_loop(0, NPAIRS, pair, 0)
    plsc.subcore_barrier()
    pltpu.sync_copy(agg_sh.at[pl.ds(s * OUT_PT, OUT_PT)],
                    out_hbm.at[c, pl.ds(s * OUT_PT, OUT_PT)])


@functools.lru_cache(maxsize=None)
def _agg_call():
    return pl.kernel(
        _agg_body,
        out_type=jax.ShapeDtypeStruct((NC, HALF, DW), jnp.float32),
        mesh=_mesh(),
        scratch_types=[
            pltpu.VMEM((CHUNKS, K), jnp.int32),
            pltpu.VMEM((CHUNKS, K), jnp.int32),
            pltpu.VMEM((K, DW), jnp.float32),
            pltpu.VMEM((K, DW), jnp.float32),
            pltpu.SemaphoreType.DMA,
            pltpu.SemaphoreType.DMA,
            pltpu.VMEM_SHARED((TBL, DW), jnp.float32),
        ],
    )


def _mm(a, b):
    return jnp.dot(a, b)


def _rsqrt(x):
    r = lax.rsqrt(x)
    return r * (1.5 - 0.5 * x * r * r)


def _grid_to_col(grid):
    si = lax.broadcasted_iota(jnp.int32, (N_PAD, DEG_ROWS), 0)
    ri = lax.broadcasted_iota(jnp.int32, (N_PAD, DEG_ROWS), 1)
    p = jnp.where(lax.shift_right_logical(si, 7) == ri, 1.0, 0.0)
    sj = lax.broadcasted_iota(jnp.int32, (N_PAD, DW), 0)
    lj = lax.broadcasted_iota(jnp.int32, (N_PAD, DW), 1)
    q = jnp.where((sj & 127) == lj, 1.0, 0.0)
    rep = _mm(p, grid)
    col = jnp.sum(rep * q, axis=1, keepdims=True)
    return col[0:N]


def _tc1_body(feat, w1, b1, w2, b2, cnt, h1s, normo):
    x = jnp.maximum(_mm(feat[:], w1[:]) + b1[:], 0.0)
    x = jnp.maximum(_mm(x, w2[:]) + b2[:], 0.0)
    dsrc = _grid_to_col(cnt[0] + cnt[1])
    no = _rsqrt(jnp.maximum(dsrc, 1.0))
    h1s[:, 0:H2] = x * no
    h1s[:, H2:DW] = jnp.zeros((N, DW - H2), jnp.float32)
    h1s[:, DW - 1:DW] = jnp.ones((N, 1), jnp.float32)
    normo[:] = no


_tc1_call = pl.pallas_call(
    _tc1_body,
    out_shape=[
        jax.ShapeDtypeStruct((N, DW), jnp.float32),
        jax.ShapeDtypeStruct((N, 1), jnp.float32),
    ],
)


def _tc2_body(agg, wc, bc, g, b, no, h2s, normi):
    degi = jnp.concatenate(
        [agg[0, :, DW - 1:DW], agg[1, 0:N - HALF, DW - 1:DW]], axis=0)
    ni = _rsqrt(jnp.maximum(degi, 1.0))
    am = jnp.concatenate([agg[0, :, 0:H2], agg[1, 0:N - HALF, 0:H2]], axis=0)
    a = am * ni
    y = _mm(a, wc[:]) + bc[:]
    mu = jnp.mean(y, axis=0, keepdims=True)
    var = jnp.mean((y - mu) ** 2, axis=0, keepdims=True)
    ybn = (y - mu) * _rsqrt(var + 1e-5) * g[:] + b[:]
    h2s[:] = jnp.maximum(ybn, 0.0) * no[:]
    normi[:] = ni


_tc2_call = pl.pallas_call(
    _tc2_body,
    out_shape=[
        jax.ShapeDtypeStruct((N, G1), jnp.float32),
        jax.ShapeDtypeStruct((N, 1), jnp.float32),
    ],
)


def _tc3_body(agg, wc, bc, g, b, ni, wd1, bd1, wd2, bd2, out):
    am = jnp.concatenate([agg[0], agg[1, 0:N - HALF, :]], axis=0)
    a = am * ni[:]
    z = _mm(a, wc[:]) + bc[:]
    mu = jnp.mean(z, axis=0, keepdims=True)
    var = jnp.mean((z - mu) ** 2, axis=0, keepdims=True)
    zbn = (z - mu) * _rsqrt(var + 1e-5) * g[:] + b[:]
    zr = jnp.maximum(zbn, 0.0)
    pooled = jnp.mean(zr, axis=0, keepdims=True)
    t = jnp.maximum(_mm(pooled, wd1[:]) + bd1[:], 0.0)
    out[:] = _mm(t, wd2[:]) + bd2[:]


_tc3_call = pl.pallas_call(
    _tc3_body,
    out_shape=jax.ShapeDtypeStruct((1, 1), jnp.float32),
)


def kernel(feat, edge_index, W_enc1, b_enc1, W_enc2, b_enc2, W_conv1, b_conv1,
           W_conv2, b_conv2, bn1_g, bn1_b, bn2_g, bn2_b, W_dec1, b_dec1,
           W_dec2, b_dec2):
    src3d = edge_index[0].reshape(NS, CHUNKS, K)
    dst3d = edge_index[1].reshape(NS, CHUNKS, K)
    src_deg = edge_index[0].reshape(NW, DEG_CPT, K)
    rowid = jnp.arange(DEG_ROWS, dtype=jnp.int32).reshape(1, DEG_ROWS)
    zeros_deg = jnp.zeros((DEG_ROWS, DW), jnp.float32)
    zeros_agg = jnp.zeros((ZRO_PT, DW), jnp.float32)

    cnt = _deg_call()(src_deg, rowid, zeros_deg)
    h1s, normo = _tc1_call(feat, W_enc1, b_enc1, W_enc2, b_enc2, cnt)
    agg1 = _agg_call()(h1s, src3d, dst3d, zeros_agg)
    h2s, normi = _tc2_call(agg1, W_conv1, b_conv1, bn1_g, bn1_b, normo)
    agg2 = _agg_call()(h2s, src3d, dst3d, zeros_agg)
    out = _tc3_call(agg2, W_conv2, b_conv2, bn2_g, bn2_b, normi,
                    W_dec1, b_dec1, W_dec2, b_dec2)
    return out
